# baseline (speedup 1.0000x reference)
import functools

import jax
import jax.numpy as jnp
from jax import lax
from jax.experimental import pallas as pl
from jax.experimental.pallas import tpu as pltpu
from jax.experimental.pallas import tpu_sc as plsc

N = 10000
E = 320000
D = 128
R = 8
B = 4096

NC = 2
NS = 16
NW = NC * NS
DH = D // NC

K = 128
CH = 157
EPT = CH * K
EPAD = NS * EPT
NPAD = 10240
RPT = NPAD // NS

BN = 512
NB = (N + BN - 1) // BN


@functools.cache
def _sc_mesh():
    return plsc.VectorSubcoreMesh(
        core_axis_name="c", subcore_axis_name="s", num_cores=NC, num_subcores=NS)


def _gidx_body(src_ref, typ_ref, out_ref):
    out_ref[...] = typ_ref[...] * N + src_ref[...]


_gidx = pl.pallas_call(
    _gidx_body,
    out_shape=jax.ShapeDtypeStruct((NS * CH, K), jnp.int32),
)


def _split_write(hrel_ref, r, res):
    hrel_ref[0, r] = res[:, :DH]
    hrel_ref[1, r] = res[:, DH:]


def _dense_body(h_ref, wrel_ref, wself_ref, hrel_ref, hself_ref):
    h = h_ref[...]
    for r in range(R):
        _split_write(hrel_ref, r, jnp.dot(h, wrel_ref[r], preferred_element_type=jnp.float32))
    hself_ref[...] = jnp.dot(h, wself_ref[...], preferred_element_type=jnp.float32)


_dense = pl.pallas_call(
    _dense_body,
    grid=(NB,),
    in_specs=[
        pl.BlockSpec((BN, D), lambda i: (i, 0)),
        pl.BlockSpec((R, D, D), lambda i: (0, 0, 0)),
        pl.BlockSpec((D, D), lambda i: (0, 0)),
    ],
    out_specs=[
        pl.BlockSpec((NC, R, BN, DH), lambda i: (0, 0, i, 0)),
        pl.BlockSpec((BN, D), lambda i: (i, 0)),
    ],
    out_shape=[
        jax.ShapeDtypeStruct((NC, R, N, DH), jnp.float32),
        jax.ShapeDtypeStruct((N, D), jnp.float32),
    ],
)


def _combine_dense_body(p_ref, degp_ref, hself_ref, b_ref, wrel_ref, wself_ref,
                        hrel_ref, hself_out_ref, degsum_ref):
    deg = jnp.maximum(jnp.sum(degp_ref[...], axis=0), 1.0)
    degsum_ref[...] = deg
    agg = jnp.concatenate([p_ref[0], p_ref[1]], axis=1)
    h = jnp.maximum(agg / deg[:, None] + hself_ref[...] + b_ref[...], 0.0)
    for r in range(R):
        _split_write(hrel_ref, r, jnp.dot(h, wrel_ref[r], preferred_element_type=jnp.float32))
    hself_out_ref[...] = jnp.dot(h, wself_ref[...], preferred_element_type=jnp.float32)


_combine_dense = pl.pallas_call(
    _combine_dense_body,
    grid=(NB,),
    in_specs=[
        pl.BlockSpec((NC, BN, DH), lambda i: (0, i, 0)),
        pl.BlockSpec((NS, BN), lambda i: (0, i)),
        pl.BlockSpec((BN, D), lambda i: (i, 0)),
        pl.BlockSpec((1, D), lambda i: (0, 0)),
        pl.BlockSpec((R, D, D), lambda i: (0, 0, 0)),
        pl.BlockSpec((D, D), lambda i: (0, 0)),
    ],
    out_specs=[
        pl.BlockSpec((NC, R, BN, DH), lambda i: (0, 0, i, 0)),
        pl.BlockSpec((BN, D), lambda i: (i, 0)),
        pl.BlockSpec((BN,), lambda i: (i,)),
    ],
    out_shape=[
        jax.ShapeDtypeStruct((NC, R, N, DH), jnp.float32),
        jax.ShapeDtypeStruct((N, D), jnp.float32),
        jax.ShapeDtypeStruct((NPAD,), jnp.float32),
    ],
)


def _mlp_body(gph_ref, ghs_ref, gdeg_ref, brg_ref, w1_ref, b1_ref,
              gm_ref, bt_ref, w2_ref, b2_ref, out_ref):
    agg = jnp.concatenate([gph_ref[0], gph_ref[1]], axis=2)
    hn = agg / gdeg_ref[...][:, :, None] + ghs_ref[...] + brg_ref[...]
    hn = jnp.maximum(hn, 0.0)
    pooled = hn[:, 0, :] + hn[:, 1, :]
    h = jnp.dot(pooled, w1_ref[...], preferred_element_type=jnp.float32) + b1_ref[...]
    mean = jnp.mean(h, axis=0, keepdims=True)
    var = jnp.mean((h - mean) ** 2, axis=0, keepdims=True)
    h = (h - mean) * lax.rsqrt(var + 1e-5) * gm_ref[...] + bt_ref[...]
    h = jnp.maximum(h, 0.0)
    out_ref[...] = jnp.dot(h, w2_ref[...], preferred_element_type=jnp.float32) + b2_ref[...]


_mlp = pl.pallas_call(
    _mlp_body,
    out_shape=jax.ShapeDtypeStruct((B, D), jnp.float32),
)


IBT = 2 * B // NS
IBC = IBT // K


@functools.cache
def _make_edge_kernel(readout):
    if readout:
        out_type = [
            jax.ShapeDtypeStruct((NC, 2 * B, DH), jnp.float32),
            jax.ShapeDtypeStruct((2 * B, D), jnp.float32),
            jax.ShapeDtypeStruct((NS, IBC, K), jnp.float32),
        ]
    else:
        out_type = [
            jax.ShapeDtypeStruct((NC, NPAD, DH), jnp.float32),
            jax.ShapeDtypeStruct((NS, NPAD), jnp.float32),
        ]

    scratch = [
        pltpu.VMEM((CH, K), jnp.int32),
        pltpu.VMEM((CH, K), jnp.int32),
        pltpu.VMEM((2, K, DH), jnp.float32),
        pltpu.VMEM((NPAD,), jnp.float32),
        pltpu.VMEM_SHARED((NPAD, DH), jnp.float32),
        pltpu.SemaphoreType.DMA,
    ]
    if readout:
        scratch += [
            pltpu.VMEM((IBC, K), jnp.int32),
            pltpu.VMEM((K, D), jnp.float32),
            pltpu.VMEM((IBC, K), jnp.float32),
        ]

    def body(hrel, gidxm, dstm, *refs):
        if readout:
            (gph, ghs, gdeg, hself_hbm, degsum_hbm, rids,
             gidxb, dstb, rows, degl, agg, sem, idb, rhs, gdegb) = (
                refs[3], refs[4], refs[5], refs[0], refs[1], refs[2],
                *refs[6:])
        else:
            parts, deg_out = refs[0], refs[1]
            gidxb, dstb, rows, degl, agg, sem = refs[2:]

        c = lax.axis_index("c")
        s = lax.axis_index("s")
        hrel_c = hrel.at[c]

        pltpu.sync_copy(gidxm.at[s], gidxb)
        pltpu.sync_copy(dstm.at[s], dstb)

        zeros16 = jnp.zeros((16,), jnp.float32)

        def zrow(r, carry):
            for k in range(DH // 16):
                rows[0, r, pl.ds(k * 16, 16)] = zeros16
            return carry
        lax.fori_loop(0, K, zrow, 0)

        base = s * RPT
        for j in range(RPT // K):
            pltpu.sync_copy(rows.at[0], agg.at[pl.ds(base + j * K, K)])

        if not readout:
            @pl.when(c == 0)
            def _():
                def zdeg(i, carry):
                    degl[pl.ds(i * 16, 16)] = zeros16
                    return carry
                lax.fori_loop(0, NPAD // 16, zdeg, 0)

        plsc.subcore_barrier()

        ones16 = jnp.ones((16,), jnp.float32)
        pltpu.async_copy(hrel_c.at[gidxb.at[0]], rows.at[0], sem)

        def main_body(j, carry):
            cur = lax.rem(j, 2)
            nxt = 1 - cur
            pltpu.make_async_copy(hrel_c.at[gidxb.at[j]], rows.at[cur], sem).wait()

            @pl.when(j < CH - 1)
            def _():
                pltpu.async_copy(hrel_c.at[gidxb.at[j + 1]], rows.at[nxt], sem)

            pltpu.sync_copy(rows.at[cur], agg.at[dstb.at[j]], add=True)
            if not readout:
                @pl.when(c == 0)
                def _():
                    for k in range(8):
                        plsc.addupdate_scatter(degl, [dstb[j, pl.ds(k * 16, 16)]], ones16)
            return carry
        lax.fori_loop(0, CH, main_body, 0)

        plsc.subcore_barrier()

        if not readout:
            for j in range(RPT // K):
                sl = pl.ds(base + j * K, K)
                pltpu.sync_copy(agg.at[sl], parts.at[c, sl])

            @pl.when(c == 0)
            def _():
                pltpu.sync_copy(degl, deg_out.at[s])
        else:
            pltpu.sync_copy(rids.at[s], idb)

            @pl.when(c == 1)
            def _():
                pltpu.sync_copy(degsum_hbm, degl)
            for j in range(IBC):
                osl = pl.ds(s * IBT + j * K, K)
                pltpu.async_copy(agg.at[idb.at[j]], rows.at[0], sem).wait()
                pltpu.sync_copy(rows.at[0], gph.at[c, osl])

                @pl.when(c == 0)
                def _():
                    pltpu.async_copy(hself_hbm.at[idb.at[j]], rhs, sem).wait()
                    pltpu.sync_copy(rhs, ghs.at[osl])

                @pl.when(c == 1)
                def _():
                    for k in range(K // 16):
                        ksl = pl.ds(k * 16, 16)
                        gdegb[j, ksl] = plsc.load_gather(degl, [idb[j, ksl]])

            @pl.when(c == 1)
            def _():
                pltpu.sync_copy(gdegb, gdeg.at[s])

    return pl.kernel(
        body,
        out_type=out_type,
        mesh=_sc_mesh(),
        compiler_params=pltpu.CompilerParams(needs_layout_passes=False, use_tc_tiling_on_sc=False),
        scratch_types=scratch,
    )


def kernel(edge_index, edge_type, node_ids, node_emb, W_rel, W_self, b_rgcn,
           W1, b1, gamma, beta, W2, b2):
    pad = EPAD - E
    srcm = jnp.pad(edge_index[0], (0, pad)).reshape(NS * CH, K)
    typm = jnp.pad(edge_type, (0, pad)).reshape(NS * CH, K)
    dstm = jnp.pad(edge_index[1], (0, pad), constant_values=N).reshape(NS, CH, K)
    rids = node_ids.reshape(NS, IBC, K)

    gidxm = _gidx(srcm, typm).reshape(NS, CH, K)

    hrel, hself = _dense(node_emb, W_rel[0], W_self[0])
    parts, degp = _make_edge_kernel(False)(hrel.reshape(NC, R * N, DH), gidxm, dstm)
    hrel, hself, degsum = _combine_dense(parts, degp, hself, b_rgcn[0:1],
                                         W_rel[1], W_self[1])
    gph, ghs, gdeg = _make_edge_kernel(True)(hrel.reshape(NC, R * N, DH),
                                             gidxm, dstm, hself, degsum, rids)
    out = _mlp(gph.reshape(NC, B, 2, DH), ghs.reshape(B, 2, D),
               gdeg.reshape(B, 2), b_rgcn[1].reshape(1, 1, D),
               W1, b1.reshape(1, 2 * D), gamma.reshape(1, 2 * D),
               beta.reshape(1, 2 * D), W2, b2.reshape(1, D))
    return out


# speedup vs baseline: 1.3376x; 1.3376x over previous
import functools

import jax
import jax.numpy as jnp
from jax import lax
from jax.experimental import pallas as pl
from jax.experimental.pallas import tpu as pltpu
from jax.experimental.pallas import tpu_sc as plsc

N = 10000
E = 320000
D = 128
R = 8
B = 4096

NC = 2
NS = 16
NW = NC * NS
DH = D // NC

K = 128
CH = 157
EPT = CH * K
EPAD = NS * EPT
NPAD = 10240
RPT = NPAD // NS

BN = 512
NB = (N + BN - 1) // BN


@functools.cache
def _sc_mesh():
    return plsc.VectorSubcoreMesh(
        core_axis_name="c", subcore_axis_name="s", num_cores=NC, num_subcores=NS)


XB = 10240


def _gidx_body(src_ref, typ_ref, out_ref):
    src = src_ref[...]
    j = jnp.bitwise_and(src, 511)
    out_ref[...] = (typ_ref[...] * XB + jnp.bitwise_and(src, ~511)
                    + 2 * jnp.bitwise_and(j, 255) + (j >> 8))


_gidx = pl.pallas_call(
    _gidx_body,
    out_shape=jax.ShapeDtypeStruct((NS * CH, K), jnp.int32),
)


def _split_write(hrel_ref, r, res):
    left, right = res[:, :DH], res[:, DH:]
    hrel_ref[0, r] = jnp.concatenate([left[:BN // 2], left[BN // 2:]], axis=1)
    hrel_ref[1, r] = jnp.concatenate([right[:BN // 2], right[BN // 2:]], axis=1)


def _dense_body(h_ref, wrel_ref, wself_ref, hrel_ref, hself_ref):
    h = h_ref[...]
    for r in range(R):
        _split_write(hrel_ref, r, jnp.dot(h, wrel_ref[r], preferred_element_type=jnp.float32))
    hself_ref[...] = jnp.dot(h, wself_ref[...], preferred_element_type=jnp.float32)


_dense = pl.pallas_call(
    _dense_body,
    grid=(NB,),
    in_specs=[
        pl.BlockSpec((BN, D), lambda i: (i, 0)),
        pl.BlockSpec((R, D, D), lambda i: (0, 0, 0)),
        pl.BlockSpec((D, D), lambda i: (0, 0)),
    ],
    out_specs=[
        pl.BlockSpec((NC, R, BN // 2, K), lambda i: (0, 0, i, 0)),
        pl.BlockSpec((BN, D), lambda i: (i, 0)),
    ],
    out_shape=[
        jax.ShapeDtypeStruct((NC, R, XB // 2, K), jnp.float32),
        jax.ShapeDtypeStruct((N, D), jnp.float32),
    ],
)


def _combine_dense_body(p_ref, degp_ref, hself_ref, b_ref, wrel_ref, wself_ref,
                        hrel_ref, hself_out_ref, degsum_ref):
    deg = jnp.maximum(jnp.sum(degp_ref[...], axis=0), 1.0)
    degsum_ref[...] = deg
    agg = jnp.concatenate([p_ref[0], p_ref[1]], axis=1)
    h = jnp.maximum(agg / deg[:, None] + hself_ref[...] + b_ref[...], 0.0)
    for r in range(R):
        _split_write(hrel_ref, r, jnp.dot(h, wrel_ref[r], preferred_element_type=jnp.float32))
    hself_out_ref[...] = jnp.dot(h, wself_ref[...], preferred_element_type=jnp.float32)


_combine_dense = pl.pallas_call(
    _combine_dense_body,
    grid=(NB,),
    in_specs=[
        pl.BlockSpec((NC, BN, DH), lambda i: (0, i, 0)),
        pl.BlockSpec((NS, BN), lambda i: (0, i)),
        pl.BlockSpec((BN, D), lambda i: (i, 0)),
        pl.BlockSpec((1, D), lambda i: (0, 0)),
        pl.BlockSpec((R, D, D), lambda i: (0, 0, 0)),
        pl.BlockSpec((D, D), lambda i: (0, 0)),
    ],
    out_specs=[
        pl.BlockSpec((NC, R, BN // 2, K), lambda i: (0, 0, i, 0)),
        pl.BlockSpec((BN, D), lambda i: (i, 0)),
        pl.BlockSpec((BN,), lambda i: (i,)),
    ],
    out_shape=[
        jax.ShapeDtypeStruct((NC, R, XB // 2, K), jnp.float32),
        jax.ShapeDtypeStruct((N, D), jnp.float32),
        jax.ShapeDtypeStruct((NPAD,), jnp.float32),
    ],
)


def _mlp_body(gph_ref, ghsa_ref, ghsb_ref, gdeg_ref, brg_ref, w1_ref, b1_ref,
              gm_ref, bt_ref, w2_ref, b2_ref, out_ref):
    g0, g1 = gph_ref[0], gph_ref[1]
    agg_a = jnp.concatenate([g0[:, :DH], g1[:, :DH]], axis=1)
    agg_b = jnp.concatenate([g0[:, DH:], g1[:, DH:]], axis=1)
    hn_a = jnp.maximum(agg_a / gdeg_ref[:, 0:1] + ghsa_ref[...] + brg_ref[...], 0.0)
    hn_b = jnp.maximum(agg_b / gdeg_ref[:, 1:2] + ghsb_ref[...] + brg_ref[...], 0.0)
    pooled = hn_a + hn_b
    h = jnp.dot(pooled, w1_ref[...], preferred_element_type=jnp.float32) + b1_ref[...]
    mean = jnp.mean(h, axis=0, keepdims=True)
    var = jnp.mean((h - mean) ** 2, axis=0, keepdims=True)
    h = (h - mean) * lax.rsqrt(var + 1e-5) * gm_ref[...] + bt_ref[...]
    h = jnp.maximum(h, 0.0)
    out_ref[...] = jnp.dot(h, w2_ref[...], preferred_element_type=jnp.float32) + b2_ref[...]


_mlp = pl.pallas_call(
    _mlp_body,
    out_shape=jax.ShapeDtypeStruct((B, D), jnp.float32),
)


IBT = 2 * B // NS
IBC = IBT // K


@functools.cache
def _make_edge_kernel(readout):
    if readout:
        out_type = [
            jax.ShapeDtypeStruct((NC, 2 * B, DH), jnp.float32),
            jax.ShapeDtypeStruct((2 * B, D), jnp.float32),
            jax.ShapeDtypeStruct((NS, IBC, K), jnp.float32),
        ]
    else:
        out_type = [
            jax.ShapeDtypeStruct((NC, NPAD, DH), jnp.float32),
            jax.ShapeDtypeStruct((NS, NPAD), jnp.float32),
        ]

    scratch = [
        pltpu.VMEM((CH, K), jnp.int32),
        pltpu.VMEM((CH, K), jnp.int32),
        pltpu.VMEM((2, K, DH), jnp.float32),
        pltpu.VMEM((NPAD,), jnp.float32),
        pltpu.VMEM_SHARED((NPAD, DH), jnp.float32),
        pltpu.SemaphoreType.DMA,
    ]
    if readout:
        scratch += [
            pltpu.VMEM((IBC, K), jnp.int32),
            pltpu.VMEM((K, D), jnp.float32),
            pltpu.VMEM((IBC, K), jnp.float32),
        ]

    def body(hrel, gidxm, dstm, *refs):
        if readout:
            (gph, ghs, gdeg, hself_hbm, degsum_hbm, rids,
             gidxb, dstb, rows, degl, agg, sem, idb, rhs, gdegb) = (
                refs[3], refs[4], refs[5], refs[0], refs[1], refs[2],
                *refs[6:])
        else:
            parts, deg_out = refs[0], refs[1]
            gidxb, dstb, rows, degl, agg, sem = refs[2:]

        c = lax.axis_index("c")
        s = lax.axis_index("s")
        hrel_c = hrel.at[c]

        pltpu.sync_copy(gidxm.at[s], gidxb)
        pltpu.sync_copy(dstm.at[s], dstb)

        zeros16 = jnp.zeros((16,), jnp.float32)

        def zrow(r, carry):
            for k in range(DH // 16):
                rows[0, r, pl.ds(k * 16, 16)] = zeros16
            return carry
        lax.fori_loop(0, K, zrow, 0)

        base = s * RPT
        for j in range(RPT // K):
            pltpu.sync_copy(rows.at[0], agg.at[pl.ds(base + j * K, K)])

        if not readout:
            @pl.when(c == 0)
            def _():
                def zdeg(i, carry):
                    degl[pl.ds(i * 16, 16)] = zeros16
                    return carry
                lax.fori_loop(0, NPAD // 16, zdeg, 0)

        plsc.subcore_barrier()

        ones16 = jnp.ones((16,), jnp.float32)
        pltpu.async_copy(hrel_c.at[gidxb.at[0]], rows.at[0], sem)

        def main_body(j, carry):
            cur = lax.rem(j, 2)
            nxt = 1 - cur
            pltpu.make_async_copy(hrel_c.at[gidxb.at[j]], rows.at[cur], sem).wait()

            @pl.when(j < CH - 1)
            def _():
                pltpu.async_copy(hrel_c.at[gidxb.at[j + 1]], rows.at[nxt], sem)

            pltpu.sync_copy(rows.at[cur], agg.at[dstb.at[j]], add=True)
            if not readout:
                @pl.when(c == 0)
                def _():
                    for k in range(8):
                        plsc.addupdate_scatter(degl, [dstb[j, pl.ds(k * 16, 16)]], ones16)
            return carry
        lax.fori_loop(0, CH, main_body, 0)

        plsc.subcore_barrier()

        if not readout:
            for j in range(RPT // K):
                sl = pl.ds(base + j * K, K)
                pltpu.sync_copy(agg.at[sl], parts.at[c, sl])

            @pl.when(c == 0)
            def _():
                pltpu.sync_copy(degl, deg_out.at[s])
        else:
            pltpu.sync_copy(rids.at[s], idb)

            @pl.when(c == 1)
            def _():
                pltpu.sync_copy(degsum_hbm, degl)
            for j in range(IBC):
                osl = pl.ds(s * IBT + j * K, K)
                pltpu.async_copy(agg.at[idb.at[j]], rows.at[0], sem).wait()
                pltpu.sync_copy(rows.at[0], gph.at[c, osl])

                @pl.when(c == 0)
                def _():
                    pltpu.async_copy(hself_hbm.at[idb.at[j]], rhs, sem).wait()
                    pltpu.sync_copy(rhs, ghs.at[osl])

                @pl.when(c == 1)
                def _():
                    for k in range(K // 16):
                        ksl = pl.ds(k * 16, 16)
                        gdegb[j, ksl] = plsc.load_gather(degl, [idb[j, ksl]])

            @pl.when(c == 1)
            def _():
                pltpu.sync_copy(gdegb, gdeg.at[s])

    return pl.kernel(
        body,
        out_type=out_type,
        mesh=_sc_mesh(),
        compiler_params=pltpu.CompilerParams(needs_layout_passes=False, use_tc_tiling_on_sc=False),
        scratch_types=scratch,
    )


def kernel(edge_index, edge_type, node_ids, node_emb, W_rel, W_self, b_rgcn,
           W1, b1, gamma, beta, W2, b2):
    pad = EPAD - E
    srcm = jnp.pad(edge_index[0], (0, pad)).reshape(NS * CH, K)
    typm = jnp.pad(edge_type, (0, pad)).reshape(NS * CH, K)
    dstm = jnp.pad(edge_index[1], (0, pad), constant_values=N).reshape(NS, CH, K)
    rids = node_ids.reshape(NS, IBC, K)

    gidxm = _gidx(srcm, typm).reshape(NS, CH, K)

    hrel, hself = _dense(node_emb, W_rel[0], W_self[0])
    parts, degp = _make_edge_kernel(False)(hrel.reshape(NC, R * XB, DH), gidxm, dstm)
    hrel, hself, degsum = _combine_dense(parts, degp, hself, b_rgcn[0:1],
                                         W_rel[1], W_self[1])
    gph, ghs, gdeg = _make_edge_kernel(True)(hrel.reshape(NC, R * XB, DH),
                                             gidxm, dstm, hself, degsum, rids)
    ghsr = ghs.reshape(B, 2, D)
    out = _mlp(gph.reshape(NC, B, K), ghsr[:, 0], ghsr[:, 1],
               gdeg.reshape(B, 2), b_rgcn[1].reshape(1, D),
               W1, b1.reshape(1, 2 * D), gamma.reshape(1, 2 * D),
               beta.reshape(1, 2 * D), W2, b2.reshape(1, D))
    return out
